# baseline (speedup 1.0000x reference)
import jax
import jax.numpy as jnp
from jax import lax
from jax.experimental import pallas as pl
from jax.experimental.pallas import tpu as pltpu
from jax.experimental.pallas import tpu_sc as plsc

N = 10000
E = 160000
D = 256
H = 256
T = 128
G = 8
EPS = 1e-5

NB = 10
BLK = N // NB

F32 = jnp.float32


def _dot(a, b):
    return jnp.dot(a, b, preferred_element_type=F32)


def _sigmoid(x):
    return 1.0 / (1.0 + jnp.exp(-x))


def _onehot(batch_ref, b):
    bv = batch_ref[b]
    return (bv == lax.broadcasted_iota(jnp.int32, (G, BLK), 0)).astype(F32)


def _p1f_body(x_ref, w_ref, b_ref, g_ref, be_ref, fcat_ref, y_scr):
    s1 = jnp.zeros((1, H), F32)
    s2 = jnp.zeros((1, H), F32)
    for b in range(NB):
        sl = pl.ds(b * BLK, BLK)
        y = _dot(x_ref[sl, :], w_ref[...]) + b_ref[...]
        y_scr[sl, :] = y
        s1 = s1 + jnp.sum(y, axis=0, keepdims=True)
        s2 = s2 + jnp.sum(y * y, axis=0, keepdims=True)

    mean = s1 / N
    var = s2 / N - mean * mean
    inv = lax.rsqrt(var + EPS)

    for b in range(NB):
        sl = pl.ds(b * BLK, BLK)
        f = jnp.maximum(
            (y_scr[sl, :] - mean) * inv * g_ref[...] + be_ref[...], 0.0)
        fcat_ref[0, sl, :] = f[:, :T]
        fcat_ref[1, sl, :] = f[:, T:]


def _p1f(x, w, b, g, be):
    return pl.pallas_call(
        _p1f_body,
        out_shape=jax.ShapeDtypeStruct((2, N, T), F32),
        scratch_shapes=[pltpu.VMEM((N, H), F32)],
    )(x, w, b.reshape(1, H), g.reshape(1, H), be.reshape(1, H))


EPT = E // 16
CHK = 128
NCHKP = 80
PADE = NCHKP * CHK
NACC = N + 16
TRASH = N
RPT = 624


def _p3_body(src_hbm, dst_hbm, feat_hbm, out_hbm,
             src0, src1, dst0, dst1, rows0, rows1, acc_sh,
             semS0, semS1, semD0, semD1, semG0, semG1, semZ):
    c = lax.axis_index("c")
    s = lax.axis_index("s")

    def idx_load(k, sbuf, dbuf, semS, semD):
        pltpu.async_copy(src_hbm.at[c, s, k], sbuf, semS)
        pltpu.async_copy(dst_hbm.at[s, k], dbuf, semD)

    def idx_wait(k, sbuf, dbuf, semS, semD):
        pltpu.make_async_copy(src_hbm.at[c, s, k], sbuf, semS).wait()
        pltpu.make_async_copy(dst_hbm.at[s, k], dbuf, semD).wait()

    idx_load(0, src0, dst0, semS0, semD0)
    idx_load(1, src1, dst1, semS1, semD1)

    def zrow(i, _):
        def zcol(j, _2):
            rows0[i, pl.ds(j * 16, 16)] = jnp.zeros((16,), F32)
            return 0
        return lax.fori_loop(0, 8, zcol, 0)

    lax.fori_loop(0, CHK, zrow, 0)
    row0 = s * RPT
    for r in range(4):
        pltpu.async_copy(rows0, acc_sh.at[pl.ds(row0 + r * CHK, CHK)], semZ)
    pltpu.async_copy(rows0.at[pl.ds(0, 112)],
                     acc_sh.at[pl.ds(row0 + 4 * CHK, 112)], semZ)

    @pl.when(s == 15)
    def _():
        pltpu.async_copy(rows0.at[pl.ds(0, 16)],
                         acc_sh.at[pl.ds(16 * RPT, 16)], semZ)

    for r in range(4):
        pltpu.make_async_copy(
            rows0, acc_sh.at[pl.ds(row0 + r * CHK, CHK)], semZ).wait()
    pltpu.make_async_copy(rows0.at[pl.ds(0, 112)],
                          acc_sh.at[pl.ds(row0 + 4 * CHK, 112)], semZ).wait()

    @pl.when(s == 15)
    def _():
        pltpu.make_async_copy(rows0.at[pl.ds(0, 16)],
                              acc_sh.at[pl.ds(16 * RPT, 16)], semZ).wait()

    plsc.subcore_barrier()

    def gather(sbuf, buf, semG):
        pltpu.async_copy(feat_hbm.at[sbuf], buf, semG)

    def gwait(sbuf, buf, semG):
        pltpu.make_async_copy(feat_hbm.at[sbuf], buf, semG).wait()

    def scat(buf, dbuf):
        pltpu.sync_copy(buf, acc_sh.at[dbuf], add=True)

    idx_wait(0, src0, dst0, semS0, semD0)
    gather(src0, rows0, semG0)

    def pair(i, _):
        k = 2 * i
        idx_wait(k + 1, src1, dst1, semS1, semD1)
        gather(src1, rows1, semG1)
        gwait(src0, rows0, semG0)
        scat(rows0, dst0)

        @pl.when(k + 2 < NCHKP)
        def _():
            idx_load(k + 2, src0, dst0, semS0, semD0)
            idx_wait(k + 2, src0, dst0, semS0, semD0)
            gather(src0, rows0, semG0)

        gwait(src1, rows1, semG1)
        scat(rows1, dst1)

        @pl.when(k + 3 < NCHKP)
        def _():
            idx_load(k + 3, src1, dst1, semS1, semD1)

        return 0

    lax.fori_loop(0, NCHKP // 2, pair, 0)

    plsc.subcore_barrier()

    pltpu.sync_copy(acc_sh.at[pl.ds(row0, RPT)],
                    out_hbm.at[c, pl.ds(row0, RPT)])

    @pl.when(s == 15)
    def _():
        pltpu.sync_copy(acc_sh.at[pl.ds(16 * RPT, 16)],
                        out_hbm.at[c, pl.ds(16 * RPT, 16)])


def _p3(src, dst, feat2n):
    mesh = plsc.VectorSubcoreMesh(core_axis_name="c", subcore_axis_name="s")
    k = pl.kernel(
        _p3_body,
        out_type=jax.ShapeDtypeStruct((2, N, T), F32),
        mesh=mesh,
        scratch_types=[
            pltpu.VMEM((CHK,), jnp.int32),
            pltpu.VMEM((CHK,), jnp.int32),
            pltpu.VMEM((CHK,), jnp.int32),
            pltpu.VMEM((CHK,), jnp.int32),
            pltpu.VMEM((CHK, T), F32),
            pltpu.VMEM((CHK, T), F32),
            pltpu.VMEM_SHARED((NACC, T), F32),
            pltpu.SemaphoreType.DMA,
            pltpu.SemaphoreType.DMA,
            pltpu.SemaphoreType.DMA,
            pltpu.SemaphoreType.DMA,
            pltpu.SemaphoreType.DMA,
            pltpu.SemaphoreType.DMA,
            pltpu.SemaphoreType.DMA,
        ],
    )
    srcp = jnp.pad(src.reshape(16, EPT), ((0, 0), (0, PADE - EPT)))
    srcs4 = jnp.stack([srcp, srcp + N]).reshape(2, 16, NCHKP, CHK)
    dstp = jnp.pad(dst.reshape(16, EPT), ((0, 0), (0, PADE - EPT)),
                   constant_values=TRASH).reshape(16, NCHKP, CHK)
    return k(srcs4, dstp, feat2n)


def _p4f_body(f_ref, a_ref, wc_ref, bc_ref, gc_ref, bec_ref,
              w0_ref, b0_ref, w1_ref, b1_ref, wl_ref, bl_ref, batch_ref,
              gm0_ref, bem0_ref, wm0_ref, bm0_ref,
              gm1_ref, bem1_ref, wm1_ref, bm1_ref,
              wsi0_ref, node0_ref, wsi1_ref, node1_ref,
              z_scr, v0_scr, v1_scr, ns0_scr):
    s1 = jnp.zeros((1, H), F32)
    s2 = jnp.zeros((1, H), F32)
    pooled = jnp.zeros((G, T), F32)
    cnt = jnp.zeros((G, T), F32)
    for b in range(NB):
        sl = pl.ds(b * BLK, BLK)
        f0 = f_ref[0, sl, :]
        f1 = f_ref[1, sl, :]
        ns0 = (_dot(f0, wl_ref[0:T, :]) + _dot(f1, wl_ref[T:H, :])
               + bl_ref[...])
        ns0_scr[sl, :] = ns0
        m = _onehot(batch_ref, b)
        pooled = pooled + lax.dot_general(
            m, ns0, (((1,), (0,)), ((), ())), preferred_element_type=F32)
        cnt = cnt + jnp.broadcast_to(
            jnp.sum(m, axis=1, keepdims=True), (G, T))
        u0 = f0 + a_ref[0, sl, :]
        u1 = f1 + a_ref[1, sl, :]
        z = _dot(u0, wc_ref[0:T, :]) + _dot(u1, wc_ref[T:H, :]) + bc_ref[...]
        z_scr[sl, :] = z
        s1 = s1 + jnp.sum(z, axis=0, keepdims=True)
        s2 = s2 + jnp.sum(z * z, axis=0, keepdims=True)

    recip = 1.0 / jnp.maximum(cnt, 1.0)
    mean = s1 / N
    var = s2 / N - mean * mean
    inv = lax.rsqrt(var + EPS)

    sa0 = jnp.zeros((1, T), F32)
    sb0 = jnp.zeros((1, T), F32)
    sa1 = jnp.zeros((1, T), F32)
    sb1 = jnp.zeros((1, T), F32)
    pb0 = jnp.zeros((G, T), F32)
    pb1 = jnp.zeros((G, T), F32)
    for b in range(NB):
        sl = pl.ds(b * BLK, BLK)
        br = jnp.maximum(
            (z_scr[sl, :] - mean) * inv * gc_ref[...] + bec_ref[...], 0.0)
        ns0 = ns0_scr[sl, :]
        m = _onehot(batch_ref, b)

        nb0 = _dot(br, w0_ref[...]) + b0_ref[...]
        v0 = ns0 + nb0
        v0_scr[sl, :] = v0
        sa0 = sa0 + jnp.sum(v0, axis=0, keepdims=True)
        sb0 = sb0 + jnp.sum(v0 * v0, axis=0, keepdims=True)
        pb0 = pb0 + lax.dot_general(
            m, nb0, (((1,), (0,)), ((), ())), preferred_element_type=F32)

        nb1 = _dot(br, w1_ref[...]) + b1_ref[...]
        v1 = ns0 + nb1
        v1_scr[sl, :] = v1
        sa1 = sa1 + jnp.sum(v1, axis=0, keepdims=True)
        sb1 = sb1 + jnp.sum(v1 * v1, axis=0, keepdims=True)
        pb1 = pb1 + lax.dot_general(
            m, nb1, (((1,), (0,)), ((), ())), preferred_element_type=F32)

    def node_head(v_scr, sa, sb, gm_ref, bem_ref, wm_ref, bm_ref, node_ref):
        meanv = sa / N
        varv = sb / N - meanv * meanv
        invv = lax.rsqrt(varv + EPS)
        for b in range(NB):
            sl = pl.ds(b * BLK, BLK)
            t = (v_scr[sl, :] - meanv) * invv * gm_ref[...] + bem_ref[...]
            node_ref[sl, :] = _sigmoid(_dot(t, wm_ref[...]) + bm_ref[...])

    node_head(v0_scr, sa0, sb0, gm0_ref, bem0_ref, wm0_ref, bm0_ref, node0_ref)
    node_head(v1_scr, sa1, sb1, gm1_ref, bem1_ref, wm1_ref, bm1_ref, node1_ref)

    def wsi_head(pb, gm_ref, bem_ref, wm_ref, bm_ref, wsi_ref):
        w = (pooled + pb) * recip
        mu = jnp.mean(w, axis=0, keepdims=True)
        var8 = jnp.mean(w * w, axis=0, keepdims=True) - mu * mu
        inv8 = lax.rsqrt(var8 + EPS)
        t = (w - mu) * inv8 * gm_ref[...] + bem_ref[...]
        wsi_ref[...] = _sigmoid(_dot(t, wm_ref[...]) + bm_ref[...])

    wsi_head(pb0, gm0_ref, bem0_ref, wm0_ref, bm0_ref, wsi0_ref)
    wsi_head(pb1, gm1_ref, bem1_ref, wm1_ref, bm1_ref, wsi1_ref)


def _p4f(fcat, agg, wc, bc, gc, bec, w0, b0, w1, b1, wl, bl, batch3d,
         gm0, bem0, wm0, bm0, gm1, bem1, wm1, bm1):
    return pl.pallas_call(
        _p4f_body,
        out_shape=[
            jax.ShapeDtypeStruct((G, 1), F32),
            jax.ShapeDtypeStruct((N, 1), F32),
            jax.ShapeDtypeStruct((G, 1), F32),
            jax.ShapeDtypeStruct((N, 1), F32),
        ],
        scratch_shapes=[
            pltpu.VMEM((N, H), F32),
            pltpu.VMEM((N, T), F32),
            pltpu.VMEM((N, T), F32),
            pltpu.VMEM((N, T), F32),
        ],
    )(fcat, agg, wc, bc.reshape(1, H), gc.reshape(1, H), bec.reshape(1, H),
      w0, b0.reshape(1, T), w1, b1.reshape(1, T), wl, bl.reshape(1, T),
      batch3d,
      gm0.reshape(1, T), bem0.reshape(1, T), wm0, bm0.reshape(1, 1),
      gm1.reshape(1, T), bem1.reshape(1, T), wm1, bm1.reshape(1, 1))


def kernel(x, edge_index, batch, W_first, b_first, g_first, be_first,
           W_lin0, b_lin0, W_conv, b_conv, g_conv, be_conv,
           W_br0, b_br0, W_br1, b_br1, g_mlp0, be_mlp0, W_mlp0, b_mlp0,
           g_mlp1, be_mlp1, W_mlp1, b_mlp1):
    src = edge_index[0]
    dst = edge_index[1]
    batch3d = batch.reshape(NB, 1, BLK)

    fcat = _p1f(x, W_first, b_first, g_first, be_first)

    feat2n = fcat.reshape(2 * N, T)
    agg = _p3(src.astype(jnp.int32), dst.astype(jnp.int32), feat2n)

    wsi0, node0, wsi1, node1 = _p4f(
        fcat, agg, W_conv, b_conv, g_conv, be_conv, W_br0, b_br0,
        W_br1, b_br1, W_lin0, b_lin0, batch3d,
        g_mlp0, be_mlp0, W_mlp0, b_mlp0, g_mlp1, be_mlp1, W_mlp1, b_mlp1)

    return (wsi0, node0, wsi1, node1)


# speedup vs baseline: 1.1991x; 1.1991x over previous
import jax
import jax.numpy as jnp
from jax import lax
from jax.experimental import pallas as pl
from jax.experimental.pallas import tpu as pltpu
from jax.experimental.pallas import tpu_sc as plsc

N = 10000
E = 160000
D = 256
H = 256
T = 128
G = 8
EPS = 1e-5

NB = 10
BLK = N // NB

F32 = jnp.float32


def _dot(a, b):
    return jnp.dot(a, b, preferred_element_type=F32)


def _sigmoid(x):
    return 1.0 / (1.0 + jnp.exp(-x))


def _onehot(batch_ref, b):
    bv = batch_ref[b]
    return (bv == lax.broadcasted_iota(jnp.int32, (G, BLK), 0)).astype(F32)


def _p1f_body(x_ref, w_ref, b_ref, g_ref, be_ref, fcat_ref, y_scr):
    s1 = jnp.zeros((1, H), F32)
    s2 = jnp.zeros((1, H), F32)
    for b in range(NB):
        sl = pl.ds(b * BLK, BLK)
        y = _dot(x_ref[sl, :], w_ref[...]) + b_ref[...]
        y_scr[sl, :] = y
        s1 = s1 + jnp.sum(y, axis=0, keepdims=True)
        s2 = s2 + jnp.sum(y * y, axis=0, keepdims=True)

    mean = s1 / N
    var = s2 / N - mean * mean
    inv = lax.rsqrt(var + EPS)

    for b in range(NB):
        sl = pl.ds(b * BLK, BLK)
        f = jnp.maximum(
            (y_scr[sl, :] - mean) * inv * g_ref[...] + be_ref[...], 0.0)
        fcat_ref[0, sl, :] = f[:, :T]
        fcat_ref[1, sl, :] = f[:, T:]


def _p1f(x, w, b, g, be):
    return pl.pallas_call(
        _p1f_body,
        out_shape=jax.ShapeDtypeStruct((2, N, T), F32),
        scratch_shapes=[pltpu.VMEM((N, H), F32)],
    )(x, w, b.reshape(1, H), g.reshape(1, H), be.reshape(1, H))


EPT = E // 16
CHK = 96
NCHKP = 105
PADE = NCHKP * CHK
NACC = N + 16
TRASH = N
RPT = 624


def _p3_body(src_hbm, dst_hbm, feat_hbm, out_hbm,
             src0, src1, src2, dst0, dst1, dst2, rows0, rows1, rows2, acc_sh,
             semS0, semS1, semS2, semD0, semD1, semD2,
             semG0, semG1, semG2, semZ):
    c = lax.axis_index("c")
    s = lax.axis_index("s")

    def idx_load(k, sbuf, dbuf, semS, semD):
        pltpu.async_copy(src_hbm.at[c, s, k], sbuf, semS)
        pltpu.async_copy(dst_hbm.at[s, k], dbuf, semD)

    def idx_wait(k, sbuf, dbuf, semS, semD):
        pltpu.make_async_copy(src_hbm.at[c, s, k], sbuf, semS).wait()
        pltpu.make_async_copy(dst_hbm.at[s, k], dbuf, semD).wait()

    idx_load(0, src0, dst0, semS0, semD0)
    idx_load(1, src1, dst1, semS1, semD1)
    idx_load(2, src2, dst2, semS2, semD2)

    def zrow(i, _):
        def zcol(j, _2):
            rows0[i, pl.ds(j * 16, 16)] = jnp.zeros((16,), F32)
            return 0
        return lax.fori_loop(0, 8, zcol, 0)

    lax.fori_loop(0, CHK, zrow, 0)
    row0 = s * RPT
    for r in range(6):
        pltpu.async_copy(rows0, acc_sh.at[pl.ds(row0 + r * CHK, CHK)], semZ)
    pltpu.async_copy(rows0.at[pl.ds(0, 48)],
                     acc_sh.at[pl.ds(row0 + 6 * CHK, 48)], semZ)

    @pl.when(s == 15)
    def _():
        pltpu.async_copy(rows0.at[pl.ds(0, 16)],
                         acc_sh.at[pl.ds(16 * RPT, 16)], semZ)

    for r in range(6):
        pltpu.make_async_copy(
            rows0, acc_sh.at[pl.ds(row0 + r * CHK, CHK)], semZ).wait()
    pltpu.make_async_copy(rows0.at[pl.ds(0, 48)],
                          acc_sh.at[pl.ds(row0 + 6 * CHK, 48)], semZ).wait()

    @pl.when(s == 15)
    def _():
        pltpu.make_async_copy(rows0.at[pl.ds(0, 16)],
                              acc_sh.at[pl.ds(16 * RPT, 16)], semZ).wait()

    plsc.subcore_barrier()

    def gather(sbuf, buf, semG):
        pltpu.async_copy(feat_hbm.at[sbuf], buf, semG)

    def gwait(sbuf, buf, semG):
        pltpu.make_async_copy(feat_hbm.at[sbuf], buf, semG).wait()

    def scat(buf, dbuf):
        pltpu.sync_copy(buf, acc_sh.at[dbuf], add=True)

    idx_wait(0, src0, dst0, semS0, semD0)
    gather(src0, rows0, semG0)
    idx_wait(1, src1, dst1, semS1, semD1)
    gather(src1, rows1, semG1)

    sets = ((src0, dst0, rows0, semS0, semD0, semG0),
            (src1, dst1, rows1, semS1, semD1, semG1),
            (src2, dst2, rows2, semS2, semD2, semG2))

    def trip(i, _):
        k = 3 * i
        for b in range(3):
            sb, db, rb, sS, sD, sG = sets[b]
            if b == 2:
                idx_wait(k + 2, sb, db, sS, sD)
                gather(sb, rb, sG)
                gwait(sb, rb, sG)
                scat(rb, db)

                @pl.when(k + 5 < NCHKP)
                def _():
                    idx_load(k + 5, sb, db, sS, sD)
            else:
                gwait(sb, rb, sG)
                scat(rb, db)

                @pl.when(k + 3 + b < NCHKP)
                def _():
                    idx_load(k + 3 + b, sb, db, sS, sD)
                    idx_wait(k + 3 + b, sb, db, sS, sD)
                    gather(sb, rb, sG)

        return 0

    lax.fori_loop(0, NCHKP // 3, trip, 0)

    plsc.subcore_barrier()

    pltpu.sync_copy(acc_sh.at[pl.ds(row0, RPT)],
                    out_hbm.at[c, pl.ds(row0, RPT)])

    @pl.when(s == 15)
    def _():
        pltpu.sync_copy(acc_sh.at[pl.ds(16 * RPT, 16)],
                        out_hbm.at[c, pl.ds(16 * RPT, 16)])


def _p3(src, dst, feat2n):
    mesh = plsc.VectorSubcoreMesh(core_axis_name="c", subcore_axis_name="s")
    k = pl.kernel(
        _p3_body,
        out_type=jax.ShapeDtypeStruct((2, N, T), F32),
        mesh=mesh,
        scratch_types=[
            pltpu.VMEM((CHK,), jnp.int32),
            pltpu.VMEM((CHK,), jnp.int32),
            pltpu.VMEM((CHK,), jnp.int32),
            pltpu.VMEM((CHK,), jnp.int32),
            pltpu.VMEM((CHK,), jnp.int32),
            pltpu.VMEM((CHK,), jnp.int32),
            pltpu.VMEM((CHK, T), F32),
            pltpu.VMEM((CHK, T), F32),
            pltpu.VMEM((CHK, T), F32),
            pltpu.VMEM_SHARED((NACC, T), F32),
            pltpu.SemaphoreType.DMA,
            pltpu.SemaphoreType.DMA,
            pltpu.SemaphoreType.DMA,
            pltpu.SemaphoreType.DMA,
            pltpu.SemaphoreType.DMA,
            pltpu.SemaphoreType.DMA,
            pltpu.SemaphoreType.DMA,
            pltpu.SemaphoreType.DMA,
            pltpu.SemaphoreType.DMA,
            pltpu.SemaphoreType.DMA,
        ],
    )
    srcp = jnp.pad(src.reshape(16, EPT), ((0, 0), (0, PADE - EPT)))
    srcs4 = jnp.stack([srcp, srcp + N]).reshape(2, 16, NCHKP, CHK)
    dstp = jnp.pad(dst.reshape(16, EPT), ((0, 0), (0, PADE - EPT)),
                   constant_values=TRASH).reshape(16, NCHKP, CHK)
    return k(srcs4, dstp, feat2n)


def _p4f_body(f_ref, a_ref, wc_ref, bc_ref, gc_ref, bec_ref,
              w0_ref, b0_ref, w1_ref, b1_ref, wl_ref, bl_ref, batch_ref,
              gm0_ref, bem0_ref, wm0_ref, bm0_ref,
              gm1_ref, bem1_ref, wm1_ref, bm1_ref,
              wsi0_ref, node0_ref, wsi1_ref, node1_ref,
              z_scr, v0_scr, v1_scr, ns0_scr):
    s1 = jnp.zeros((1, H), F32)
    s2 = jnp.zeros((1, H), F32)
    pooled = jnp.zeros((G, T), F32)
    cnt = jnp.zeros((G, T), F32)
    for b in range(NB):
        sl = pl.ds(b * BLK, BLK)
        f0 = f_ref[0, sl, :]
        f1 = f_ref[1, sl, :]
        ns0 = (_dot(f0, wl_ref[0:T, :]) + _dot(f1, wl_ref[T:H, :])
               + bl_ref[...])
        ns0_scr[sl, :] = ns0
        m = _onehot(batch_ref, b)
        pooled = pooled + lax.dot_general(
            m, ns0, (((1,), (0,)), ((), ())), preferred_element_type=F32)
        cnt = cnt + jnp.broadcast_to(
            jnp.sum(m, axis=1, keepdims=True), (G, T))
        u0 = f0 + a_ref[0, sl, :]
        u1 = f1 + a_ref[1, sl, :]
        z = _dot(u0, wc_ref[0:T, :]) + _dot(u1, wc_ref[T:H, :]) + bc_ref[...]
        z_scr[sl, :] = z
        s1 = s1 + jnp.sum(z, axis=0, keepdims=True)
        s2 = s2 + jnp.sum(z * z, axis=0, keepdims=True)

    recip = 1.0 / jnp.maximum(cnt, 1.0)
    mean = s1 / N
    var = s2 / N - mean * mean
    inv = lax.rsqrt(var + EPS)

    sa0 = jnp.zeros((1, T), F32)
    sb0 = jnp.zeros((1, T), F32)
    sa1 = jnp.zeros((1, T), F32)
    sb1 = jnp.zeros((1, T), F32)
    pb0 = jnp.zeros((G, T), F32)
    pb1 = jnp.zeros((G, T), F32)
    for b in range(NB):
        sl = pl.ds(b * BLK, BLK)
        br = jnp.maximum(
            (z_scr[sl, :] - mean) * inv * gc_ref[...] + bec_ref[...], 0.0)
        ns0 = ns0_scr[sl, :]
        m = _onehot(batch_ref, b)

        nb0 = _dot(br, w0_ref[...]) + b0_ref[...]
        v0 = ns0 + nb0
        v0_scr[sl, :] = v0
        sa0 = sa0 + jnp.sum(v0, axis=0, keepdims=True)
        sb0 = sb0 + jnp.sum(v0 * v0, axis=0, keepdims=True)
        pb0 = pb0 + lax.dot_general(
            m, nb0, (((1,), (0,)), ((), ())), preferred_element_type=F32)

        nb1 = _dot(br, w1_ref[...]) + b1_ref[...]
        v1 = ns0 + nb1
        v1_scr[sl, :] = v1
        sa1 = sa1 + jnp.sum(v1, axis=0, keepdims=True)
        sb1 = sb1 + jnp.sum(v1 * v1, axis=0, keepdims=True)
        pb1 = pb1 + lax.dot_general(
            m, nb1, (((1,), (0,)), ((), ())), preferred_element_type=F32)

    def node_head(v_scr, sa, sb, gm_ref, bem_ref, wm_ref, bm_ref, node_ref):
        meanv = sa / N
        varv = sb / N - meanv * meanv
        invv = lax.rsqrt(varv + EPS)
        for b in range(NB):
            sl = pl.ds(b * BLK, BLK)
            t = (v_scr[sl, :] - meanv) * invv * gm_ref[...] + bem_ref[...]
            node_ref[sl, :] = _sigmoid(_dot(t, wm_ref[...]) + bm_ref[...])

    node_head(v0_scr, sa0, sb0, gm0_ref, bem0_ref, wm0_ref, bm0_ref, node0_ref)
    node_head(v1_scr, sa1, sb1, gm1_ref, bem1_ref, wm1_ref, bm1_ref, node1_ref)

    def wsi_head(pb, gm_ref, bem_ref, wm_ref, bm_ref, wsi_ref):
        w = (pooled + pb) * recip
        mu = jnp.mean(w, axis=0, keepdims=True)
        var8 = jnp.mean(w * w, axis=0, keepdims=True) - mu * mu
        inv8 = lax.rsqrt(var8 + EPS)
        t = (w - mu) * inv8 * gm_ref[...] + bem_ref[...]
        wsi_ref[...] = _sigmoid(_dot(t, wm_ref[...]) + bm_ref[...])

    wsi_head(pb0, gm0_ref, bem0_ref, wm0_ref, bm0_ref, wsi0_ref)
    wsi_head(pb1, gm1_ref, bem1_ref, wm1_ref, bm1_ref, wsi1_ref)


def _p4f(fcat, agg, wc, bc, gc, bec, w0, b0, w1, b1, wl, bl, batch3d,
         gm0, bem0, wm0, bm0, gm1, bem1, wm1, bm1):
    return pl.pallas_call(
        _p4f_body,
        out_shape=[
            jax.ShapeDtypeStruct((G, 1), F32),
            jax.ShapeDtypeStruct((N, 1), F32),
            jax.ShapeDtypeStruct((G, 1), F32),
            jax.ShapeDtypeStruct((N, 1), F32),
        ],
        scratch_shapes=[
            pltpu.VMEM((N, H), F32),
            pltpu.VMEM((N, T), F32),
            pltpu.VMEM((N, T), F32),
            pltpu.VMEM((N, T), F32),
        ],
    )(fcat, agg, wc, bc.reshape(1, H), gc.reshape(1, H), bec.reshape(1, H),
      w0, b0.reshape(1, T), w1, b1.reshape(1, T), wl, bl.reshape(1, T),
      batch3d,
      gm0.reshape(1, T), bem0.reshape(1, T), wm0, bm0.reshape(1, 1),
      gm1.reshape(1, T), bem1.reshape(1, T), wm1, bm1.reshape(1, 1))


def kernel(x, edge_index, batch, W_first, b_first, g_first, be_first,
           W_lin0, b_lin0, W_conv, b_conv, g_conv, be_conv,
           W_br0, b_br0, W_br1, b_br1, g_mlp0, be_mlp0, W_mlp0, b_mlp0,
           g_mlp1, be_mlp1, W_mlp1, b_mlp1):
    src = edge_index[0]
    dst = edge_index[1]
    batch3d = batch.reshape(NB, 1, BLK)

    fcat = _p1f(x, W_first, b_first, g_first, be_first)

    feat2n = fcat.reshape(2 * N, T)
    agg = _p3(src.astype(jnp.int32), dst.astype(jnp.int32), feat2n)

    wsi0, node0, wsi1, node1 = _p4f(
        fcat, agg, W_conv, b_conv, g_conv, be_conv, W_br0, b_br0,
        W_br1, b_br1, W_lin0, b_lin0, batch3d,
        g_mlp0, be_mlp0, W_mlp0, b_mlp0, g_mlp1, be_mlp1, W_mlp1, b_mlp1)

    return (wsi0, node0, wsi1, node1)
